# baseline (speedup 1.0000x reference)
import jax
import jax.numpy as jnp
from jax.experimental import pallas as pl
from jax.experimental.pallas import tpu as pltpu

_N = 10000
_INFEAT = 128
_HIDDEN = 24
_OUTFEAT = 128
_BM = 400
_NB = _N // _BM
_NS = 3


def _body(x_ref, adj_hbm, w1_ref, b1_ref, w2_ref, b2_ref, o_ref,
          s_ref, bufs, sems):
    i = pl.program_id(0)

    _Q = 5
    _CH = _BM // _Q

    def _start_block(blk, s):
        for q in range(_Q):
            pltpu.make_async_copy(
                adj_hbm.at[pl.ds(blk * _BM + q * _CH, _CH), :],
                bufs.at[s].at[pl.ds(q * _CH, _CH), :],
                sems.at[s, q]).start()

    def _wait_block(blk, s):
        for q in range(_Q):
            pltpu.make_async_copy(
                adj_hbm.at[pl.ds(blk * _BM + q * _CH, _CH), :],
                bufs.at[s].at[pl.ds(q * _CH, _CH), :],
                sems.at[s, q]).wait()

    @pl.when(i == 0)
    def _():
        for k in range(_NS):
            _start_block(k, k)
        s_ref[...] = jnp.dot(x_ref[...], w1_ref[...],
                             preferred_element_type=jnp.float32)

    def _branch(s):
        def br():
            _wait_block(i, s)
            acc = jnp.dot(bufs[s], s_ref[...],
                          preferred_element_type=jnp.float32)

            @pl.when(i + _NS < _NB)
            def _():
                _start_block(i + _NS, s)

            h = acc + b1_ref[...]
            h = jnp.where(h > 0, h, 0.01 * h)
            o_ref[...] = jnp.tanh(
                jnp.dot(h, w2_ref[...],
                        preferred_element_type=jnp.float32)
                + b2_ref[...])
        return br

    jax.lax.switch(jax.lax.rem(i, _NS),
                   [_branch(0), _branch(1), _branch(2)])


def kernel(x, adj, W1, b1, W2, b2):
    b1r = b1.reshape(1, _HIDDEN)
    b2r = b2.reshape(1, _OUTFEAT)

    return pl.pallas_call(
        _body,
        grid=(_NB,),
        in_specs=[
            pl.BlockSpec((_N, _INFEAT), lambda i: (0, 0)),
            pl.BlockSpec(memory_space=pltpu.MemorySpace.HBM),
            pl.BlockSpec((_INFEAT, _HIDDEN), lambda i: (0, 0)),
            pl.BlockSpec((1, _HIDDEN), lambda i: (0, 0)),
            pl.BlockSpec((_HIDDEN, _OUTFEAT), lambda i: (0, 0)),
            pl.BlockSpec((1, _OUTFEAT), lambda i: (0, 0)),
        ],
        out_specs=pl.BlockSpec((_BM, _OUTFEAT), lambda i: (i, 0)),
        out_shape=jax.ShapeDtypeStruct((_N, _OUTFEAT), jnp.float32),
        scratch_shapes=[
            pltpu.VMEM((_N, _HIDDEN), jnp.float32),
            pltpu.VMEM((_NS, _BM, _N), jnp.float32),
            pltpu.SemaphoreType.DMA((_NS, 5)),
        ],
        compiler_params=pltpu.CompilerParams(
            vmem_limit_bytes=64 * 1024 * 1024),
    )(x, adj, W1, b1r, W2, b2r)


# speedup vs baseline: 1.0378x; 1.0378x over previous
import jax
import jax.numpy as jnp
from jax.experimental import pallas as pl
from jax.experimental.pallas import tpu as pltpu

_N = 10000
_INFEAT = 128
_HIDDEN = 24
_OUTFEAT = 128
_BM = 400


def _body(x_ref, adj_ref, w1_ref, b1_ref, w2_ref, b2_ref, o_ref, s_ref):
    @pl.when(pl.program_id(0) == 0)
    def _():
        s_ref[...] = jnp.dot(x_ref[...], w1_ref[...],
                             preferred_element_type=jnp.float32)

    acc = jnp.dot(adj_ref[...], s_ref[...],
                  preferred_element_type=jnp.float32)
    h = acc + b1_ref[...]
    h = jnp.where(h > 0, h, 0.01 * h)
    o_ref[...] = jnp.tanh(
        jnp.dot(h, w2_ref[...], preferred_element_type=jnp.float32)
        + b2_ref[...])


def kernel(x, adj, W1, b1, W2, b2):
    b1r = b1.reshape(1, _HIDDEN)
    b2r = b2.reshape(1, _OUTFEAT)

    return pl.pallas_call(
        _body,
        grid=(_N // _BM,),
        in_specs=[
            pl.BlockSpec((_N, _INFEAT), lambda i: (0, 0)),
            pl.BlockSpec((_BM, _N), lambda i: (i, 0)),
            pl.BlockSpec((_INFEAT, _HIDDEN), lambda i: (0, 0)),
            pl.BlockSpec((1, _HIDDEN), lambda i: (0, 0)),
            pl.BlockSpec((_HIDDEN, _OUTFEAT), lambda i: (0, 0)),
            pl.BlockSpec((1, _OUTFEAT), lambda i: (0, 0)),
        ],
        out_specs=pl.BlockSpec((_BM, _OUTFEAT), lambda i: (i, 0)),
        out_shape=jax.ShapeDtypeStruct((_N, _OUTFEAT), jnp.float32),
        scratch_shapes=[pltpu.VMEM((_N, _HIDDEN), jnp.float32)],
    )(x, adj, W1, b1r, W2, b2r)
